# baseline (speedup 1.0000x reference)
import jax
import jax.numpy as jnp
from jax.experimental import pallas as pl
from jax.experimental.pallas import tpu as pltpu

_ROWS = 8192
_COLS = 4096
_BLK = 512
_N = _ROWS // _BLK


def _body(x_hbm, o_hbm, va, vb, isem, osem):
    c0 = pltpu.make_async_copy(x_hbm.at[pl.ds(0, _BLK)], va, isem)
    c0.start()
    c0.wait()
    for i in range(_N):
        ci = pltpu.make_async_copy(x_hbm.at[pl.ds(i * _BLK, _BLK)], vb, isem)
        co = pltpu.make_async_copy(va, o_hbm.at[pl.ds(i * _BLK, _BLK)], osem)
        ci.start()
        co.start()
        ci.wait()
        co.wait()


def kernel(x):
    return pl.pallas_call(
        _body,
        in_specs=[pl.BlockSpec(memory_space=pl.ANY)],
        out_specs=pl.BlockSpec(memory_space=pl.ANY),
        out_shape=jax.ShapeDtypeStruct((_ROWS, _COLS), x.dtype),
        scratch_shapes=[
            pltpu.VMEM((_BLK, _COLS), jnp.float32),
            pltpu.VMEM((_BLK, _COLS), jnp.float32),
            pltpu.SemaphoreType.DMA,
            pltpu.SemaphoreType.DMA,
        ],
    )(x)


# speedup vs baseline: 1.2978x; 1.2978x over previous
import jax
import jax.numpy as jnp
from jax.experimental import pallas as pl

_COL_A = 5
_COL_B = 1000
_ROWS = 8192
_COLS = 4096
_BLK = 512


def _swap_body(x_ref, o_ref):
    xv = x_ref[...]
    o_ref[...] = xv
    o_ref[:, _COL_A:_COL_A + 1] = xv[:, _COL_B:_COL_B + 1]
    o_ref[:, _COL_B:_COL_B + 1] = xv[:, _COL_A:_COL_A + 1]


def kernel(x):
    return pl.pallas_call(
        _swap_body,
        grid=(_ROWS // _BLK,),
        in_specs=[pl.BlockSpec((_BLK, _COLS), lambda i: (i, 0))],
        out_specs=pl.BlockSpec((_BLK, _COLS), lambda i: (i, 0)),
        out_shape=jax.ShapeDtypeStruct((_ROWS, _COLS), x.dtype),
    )(x)
